# baseline (speedup 1.0000x reference)
import functools

import jax
import jax.numpy as jnp
from jax import lax
from jax.experimental import pallas as pl
from jax.experimental.pallas import tpu as pltpu
from jax.experimental.pallas import tpu_sc as plsc

_BATCH = 16384
_D = 64
_NC = 2
_NS = 16
_NW = _NC * _NS
_PER_W = _BATCH // _NW
_C = 128
_NCHUNK = _PER_W // _C
_NG = _C // 16
_MARGIN = 1.0


def _rsqrt_nr(x):
    x = jnp.maximum(x, 1e-12)
    i = plsc.bitcast(x, jnp.int32)
    magic = jnp.full((16,), 0x5F3759DF, jnp.int32)
    y = plsc.bitcast(magic - jnp.right_shift(i, jnp.full((16,), 1, jnp.int32)),
                     jnp.float32)
    half_x = 0.5 * x
    for _ in range(3):
        y = y * (1.5 - half_x * y * y)
    return y


def _body(pos_h, pos_t, pos_r, neg_h, neg_t, neg_r, ent, rel, out,
          i_ph, i_pt, i_pr, i_nh, i_nt, i_nr,
          r_ph, r_pt, r_pr, r_nh, r_nt, r_nr,
          acc_v, sem):
    wid = lax.axis_index("s") * _NC + lax.axis_index("c")
    iota = lax.iota(jnp.int32, 16)
    def sumsq_step(d, accs):
        row = accs[-1]
        col = jnp.full((16,), d, jnp.int32)
        vals = tuple(plsc.load_gather(ref, [row, col])
                     for ref in (r_ph, r_pt, r_pr, r_nh, r_nt, r_nr))
        new = tuple(a + v * v for a, v in zip(accs[:-1], vals))
        return new + (row,)

    def score_step(d, carry):
        pacc, nacc, row, inv = carry
        ihp, itp, irp, ihn, itn, irn = inv
        col = jnp.full((16,), d, jnp.int32)
        h = plsc.load_gather(r_ph, [row, col])
        t = plsc.load_gather(r_pt, [row, col])
        r = plsc.load_gather(r_pr, [row, col])
        pacc = pacc + jnp.abs(h * ihp + r * irp - t * itp)
        h = plsc.load_gather(r_nh, [row, col])
        t = plsc.load_gather(r_nt, [row, col])
        r = plsc.load_gather(r_nr, [row, col])
        nacc = nacc + jnp.abs(h * ihn + r * irn - t * itn)
        return pacc, nacc, row, inv

    def group_body(g, total):
        row = g * 16 + iota
        z = jnp.zeros((16,), jnp.float32)
        sq = lax.fori_loop(0, _D, sumsq_step, (z, z, z, z, z, z, row))
        inv = tuple(_rsqrt_nr(s) for s in sq[:-1])
        pacc, nacc, _, _ = lax.fori_loop(0, _D, score_step, (z, z, row, inv))
        return total + jnp.maximum(pacc - nacc + _MARGIN, 0.0)

    total = jnp.zeros((16,), jnp.float32)
    for chunk in range(_NCHUNK):
        base = wid * _PER_W + chunk * _C
        pltpu.sync_copy(pos_h.at[pl.ds(base, _C)], i_ph)
        pltpu.sync_copy(pos_t.at[pl.ds(base, _C)], i_pt)
        pltpu.sync_copy(pos_r.at[pl.ds(base, _C)], i_pr)
        pltpu.sync_copy(neg_h.at[pl.ds(base, _C)], i_nh)
        pltpu.sync_copy(neg_t.at[pl.ds(base, _C)], i_nt)
        pltpu.sync_copy(neg_r.at[pl.ds(base, _C)], i_nr)
        cps = [
            pltpu.async_copy(ent.at[i_ph], r_ph, sem),
            pltpu.async_copy(ent.at[i_pt], r_pt, sem),
            pltpu.async_copy(rel.at[i_pr], r_pr, sem),
            pltpu.async_copy(ent.at[i_nh], r_nh, sem),
            pltpu.async_copy(ent.at[i_nt], r_nt, sem),
            pltpu.async_copy(rel.at[i_nr], r_nr, sem),
        ]
        for cp in cps:
            cp.wait()
        total = lax.fori_loop(0, _NG, group_body, total)

    acc_v[...] = total * (1.0 / _BATCH)
    pltpu.sync_copy(acc_v, out.at[wid])


@jax.jit
def kernel(pos_h, pos_t, pos_r, neg_h, neg_t, neg_r, ent_emb, rel_emb):
    mesh = plsc.VectorSubcoreMesh(core_axis_name="c", subcore_axis_name="s",
                                  num_cores=_NC, num_subcores=_NS)
    run = functools.partial(
        pl.kernel,
        out_type=jax.ShapeDtypeStruct((_NW, 16), jnp.float32),
        mesh=mesh,
        scratch_types=[pltpu.VMEM((_C,), jnp.int32)] * 6
        + [pltpu.VMEM((_C, _D), jnp.float32)] * 6
        + [pltpu.VMEM((16,), jnp.float32), pltpu.SemaphoreType.DMA],
        compiler_params=pltpu.CompilerParams(needs_layout_passes=False,
                                             use_tc_tiling_on_sc=False),
    )(_body)
    partial_sums = run(pos_h, pos_t, pos_r, neg_h, neg_t, neg_r,
                       ent_emb, rel_emb)
    return jnp.sum(partial_sums)


# speedup vs baseline: 2.0859x; 2.0859x over previous
import functools

import jax
import jax.numpy as jnp
from jax import lax
from jax.experimental import pallas as pl
from jax.experimental.pallas import tpu as pltpu
from jax.experimental.pallas import tpu_sc as plsc

_BATCH = 16384
_D = 64
_NC = 2
_NS = 16
_NW = _NC * _NS
_PER_W = _BATCH // _NW
_C = 128
_NCHUNK = _PER_W // _C
_NG = _C // 16
_MARGIN = 1.0
_PACK_W = 8192


def _pack_body(x, o):
    ident = jnp.eye(_D, dtype=jnp.float32)
    dn = (((0,), (0,)), ((), ()))
    for s in range(x.shape[1] // 128):
        v = x[:, s * 128:(s + 1) * 128]
        inv = lax.rsqrt(jnp.maximum(
            jnp.sum(v * v, axis=0, keepdims=True), 1e-12))
        vn = v * inv
        half = (s % 2) * _D
        r0 = (s // 2) * 128
        o[r0:r0 + 128, half:half + _D] = lax.dot_general(
            vn, ident, dn, preferred_element_type=jnp.float32)


def _pack(table_t, w):
    nblk = (table_t.shape[1] + w - 1) // w
    return pl.pallas_call(
        _pack_body,
        grid=(nblk,),
        in_specs=[pl.BlockSpec((_D, w), lambda j: (0, j))],
        out_specs=pl.BlockSpec((w // 2, 128), lambda j: (j, 0)),
        out_shape=jax.ShapeDtypeStruct((nblk * w // 2, 128), jnp.float32),
    )(table_t)


def _row_of(i):
    return jnp.left_shift(jnp.right_shift(i, 8), 7) + jnp.bitwise_and(i, 127)


def _off_of(i):
    return jnp.left_shift(jnp.bitwise_and(jnp.right_shift(i, 7), 1), 6)


def _body(pos_h, pos_t, pos_r, neg_h, neg_t, neg_r, ent_p, rel_p, out,
          i_ph, i_pt, i_pr, i_nh, i_nt, i_nr,
          a_ph, a_pt, a_pr, a_nh, a_nt, a_nr,
          r_ph, r_pt, r_pr, r_nh, r_nt, r_nr,
          acc_v, sem):
    wid = lax.axis_index("s") * _NC + lax.axis_index("c")
    iota = lax.iota(jnp.int32, 16)
    raws = (i_ph, i_pt, i_pr, i_nh, i_nt, i_nr)
    adjs = (a_ph, a_pt, a_pr, a_nh, a_nt, a_nr)

    def remap(raw_ref, adj_ref):
        def step(v, _):
            x = raw_ref[pl.ds(v * 16, 16)]
            adj_ref[pl.ds(v * 16, 16)] = _row_of(x)
            return 0

        lax.fori_loop(0, _C // 16, step, 0)

    def score_step(d, carry):
        pacc, nacc, row, offs = carry
        h = plsc.load_gather(r_ph, [row, offs[0] + d])
        t = plsc.load_gather(r_pt, [row, offs[1] + d])
        r = plsc.load_gather(r_pr, [row, offs[2] + d])
        pacc = pacc + jnp.abs(h + r - t)
        h = plsc.load_gather(r_nh, [row, offs[3] + d])
        t = plsc.load_gather(r_nt, [row, offs[4] + d])
        r = plsc.load_gather(r_nr, [row, offs[5] + d])
        nacc = nacc + jnp.abs(h + r - t)
        return pacc, nacc, row, offs

    def group_body(g, total):
        row = g * 16 + iota
        offs = tuple(_off_of(raw_ref[pl.ds(g * 16, 16)]) for raw_ref in raws)
        z = jnp.zeros((16,), jnp.float32)
        pacc, nacc, _, _ = lax.fori_loop(
            0, _D, score_step, (z, z, row, offs), unroll=8)
        return total + jnp.maximum(pacc - nacc + _MARGIN, 0.0)

    total = jnp.zeros((16,), jnp.float32)
    for chunk in range(_NCHUNK):
        base = wid * _PER_W + chunk * _C
        for src, raw_ref in zip((pos_h, pos_t, pos_r, neg_h, neg_t, neg_r),
                                raws):
            pltpu.sync_copy(src.at[pl.ds(base, _C)], raw_ref)
        for raw_ref, adj_ref in zip(raws, adjs):
            remap(raw_ref, adj_ref)
        cps = [
            pltpu.async_copy(ent_p.at[a_ph], r_ph, sem),
            pltpu.async_copy(ent_p.at[a_pt], r_pt, sem),
            pltpu.async_copy(rel_p.at[a_pr], r_pr, sem),
            pltpu.async_copy(ent_p.at[a_nh], r_nh, sem),
            pltpu.async_copy(ent_p.at[a_nt], r_nt, sem),
            pltpu.async_copy(rel_p.at[a_nr], r_nr, sem),
        ]
        for cp in cps:
            cp.wait()
        total = lax.fori_loop(0, _NG, group_body, total)

    acc_v[...] = total * (1.0 / _BATCH)
    pltpu.sync_copy(acc_v, out.at[wid])


@jax.jit
def kernel(pos_h, pos_t, pos_r, neg_h, neg_t, neg_r, ent_emb, rel_emb):
    ent_p = _pack(jnp.transpose(ent_emb), _PACK_W)
    rel_p = _pack(jnp.transpose(rel_emb), 1024)
    mesh = plsc.VectorSubcoreMesh(core_axis_name="c", subcore_axis_name="s",
                                  num_cores=_NC, num_subcores=_NS)
    run = functools.partial(
        pl.kernel,
        out_type=jax.ShapeDtypeStruct((_NW, 16), jnp.float32),
        mesh=mesh,
        scratch_types=[pltpu.VMEM((_C,), jnp.int32)] * 12
        + [pltpu.VMEM((_C, 128), jnp.float32)] * 6
        + [pltpu.VMEM((16,), jnp.float32), pltpu.SemaphoreType.DMA],
        compiler_params=pltpu.CompilerParams(needs_layout_passes=False,
                                             use_tc_tiling_on_sc=False),
    )(_body)
    partial_sums = run(pos_h, pos_t, pos_r, neg_h, neg_t, neg_r,
                       ent_p, rel_p)
    return jnp.sum(partial_sums)


# speedup vs baseline: 2.2323x; 1.0702x over previous
import functools

import jax
import jax.numpy as jnp
from jax import lax
from jax.experimental import pallas as pl
from jax.experimental.pallas import tpu as pltpu
from jax.experimental.pallas import tpu_sc as plsc

_BATCH = 16384
_D = 64
_NC = 2
_NS = 16
_NW = _NC * _NS
_PER_W = _BATCH // _NW
_C = 64
_NCHUNK = _PER_W // _C
_NG = _C // 16
_MARGIN = 1.0
_PACK_W = 8192


def _pack_body(x, o):
    ident = jnp.eye(_D, dtype=jnp.float32)
    dn = (((0,), (0,)), ((), ()))
    for s in range(x.shape[1] // 128):
        v = x[:, s * 128:(s + 1) * 128]
        inv = lax.rsqrt(jnp.maximum(
            jnp.sum(v * v, axis=0, keepdims=True), 1e-12))
        vn = v * inv
        half = (s % 2) * _D
        r0 = (s // 2) * 128
        o[r0:r0 + 128, half:half + _D] = lax.dot_general(
            vn, ident, dn, preferred_element_type=jnp.float32)


def _pack(table_t, w):
    nblk = (table_t.shape[1] + w - 1) // w
    return pl.pallas_call(
        _pack_body,
        grid=(nblk,),
        in_specs=[pl.BlockSpec((_D, w), lambda j: (0, j))],
        out_specs=pl.BlockSpec((w // 2, 128), lambda j: (j, 0)),
        out_shape=jax.ShapeDtypeStruct((nblk * w // 2, 128), jnp.float32),
    )(table_t)


def _row_of(i):
    return jnp.left_shift(jnp.right_shift(i, 8), 7) + jnp.bitwise_and(i, 127)


def _off_of(i):
    return jnp.left_shift(jnp.bitwise_and(jnp.right_shift(i, 7), 1), 6)


def _body(pos_h, pos_t, pos_r, neg_h, neg_t, neg_r, ent_p, rel_p, out,
          i_ph, i_pt, i_pr, i_nh, i_nt, i_nr,
          a_ph, a_pt, a_pr, a_nh, a_nt, a_nr,
          rA_ph, rA_pt, rA_pr, rA_nh, rA_nt, rA_nr,
          rB_ph, rB_pt, rB_pr, rB_nh, rB_nt, rB_nr,
          acc_v, semA, semB):
    wid = lax.axis_index("s") * _NC + lax.axis_index("c")
    iota = lax.iota(jnp.int32, 16)
    raws = (i_ph, i_pt, i_pr, i_nh, i_nt, i_nr)
    adjs = (a_ph, a_pt, a_pr, a_nh, a_nt, a_nr)
    bufs = ((rA_ph, rA_pt, rA_pr, rA_nh, rA_nt, rA_nr),
            (rB_ph, rB_pt, rB_pr, rB_nh, rB_nt, rB_nr))
    sems = (semA, semB)

    idx_cps = [
        pltpu.async_copy(src.at[pl.ds(wid * _PER_W, _PER_W)], raw_ref, semA)
        for src, raw_ref in zip((pos_h, pos_t, pos_r, neg_h, neg_t, neg_r),
                                raws)
    ]
    for cp in idx_cps:
        cp.wait()

    def remap(raw_ref, adj_ref):
        def step(v, _):
            x = raw_ref[pl.ds(v * 16, 16)]
            adj_ref[pl.ds(v * 16, 16)] = _row_of(x)
            return 0

        lax.fori_loop(0, _PER_W // 16, step, 0)

    for raw_ref, adj_ref in zip(raws, adjs):
        remap(raw_ref, adj_ref)

    def fire(k):
        sl = pl.ds(k * _C, _C)
        b = bufs[k % 2]
        sem = sems[k % 2]
        return [
            pltpu.async_copy(ent_p.at[a_ph.at[sl]], b[0], sem),
            pltpu.async_copy(ent_p.at[a_pt.at[sl]], b[1], sem),
            pltpu.async_copy(rel_p.at[a_pr.at[sl]], b[2], sem),
            pltpu.async_copy(ent_p.at[a_nh.at[sl]], b[3], sem),
            pltpu.async_copy(ent_p.at[a_nt.at[sl]], b[4], sem),
            pltpu.async_copy(rel_p.at[a_nr.at[sl]], b[5], sem),
        ]

    def make_score_step(b):
        def score_step(d, carry):
            pacc, nacc, row, offs = carry
            h = plsc.load_gather(b[0], [row, offs[0] + d])
            t = plsc.load_gather(b[1], [row, offs[1] + d])
            r = plsc.load_gather(b[2], [row, offs[2] + d])
            pacc = pacc + jnp.abs(h + r - t)
            h = plsc.load_gather(b[3], [row, offs[3] + d])
            t = plsc.load_gather(b[4], [row, offs[4] + d])
            r = plsc.load_gather(b[5], [row, offs[5] + d])
            nacc = nacc + jnp.abs(h + r - t)
            return pacc, nacc, row, offs

        return score_step

    def compute(k, total):
        b = bufs[k % 2]
        step = make_score_step(b)

        def group_body(g, tot):
            row = g * 16 + iota
            offs = tuple(_off_of(raw_ref[pl.ds(k * _C + g * 16, 16)])
                         for raw_ref in raws)
            z = jnp.zeros((16,), jnp.float32)
            pacc, nacc, _, _ = lax.fori_loop(
                0, _D, step, (z, z, row, offs), unroll=8)
            return tot + jnp.maximum(pacc - nacc + _MARGIN, 0.0)

        return lax.fori_loop(0, _NG, group_body, total)

    total = jnp.zeros((16,), jnp.float32)
    cps = fire(0)
    for k in range(_NCHUNK):
        nxt = fire(k + 1) if k + 1 < _NCHUNK else []
        for cp in cps:
            cp.wait()
        total = compute(k, total)
        cps = nxt

    acc_v[...] = total * (1.0 / _BATCH)
    pltpu.sync_copy(acc_v, out.at[wid])


@jax.jit
def kernel(pos_h, pos_t, pos_r, neg_h, neg_t, neg_r, ent_emb, rel_emb):
    ent_p = _pack(jnp.transpose(ent_emb), _PACK_W)
    rel_p = _pack(jnp.transpose(rel_emb), 1024)
    mesh = plsc.VectorSubcoreMesh(core_axis_name="c", subcore_axis_name="s",
                                  num_cores=_NC, num_subcores=_NS)
    run = functools.partial(
        pl.kernel,
        out_type=jax.ShapeDtypeStruct((_NW, 16), jnp.float32),
        mesh=mesh,
        scratch_types=[pltpu.VMEM((_PER_W,), jnp.int32)] * 12
        + [pltpu.VMEM((_C, 128), jnp.float32)] * 12
        + [pltpu.VMEM((16,), jnp.float32),
           pltpu.SemaphoreType.DMA, pltpu.SemaphoreType.DMA],
        compiler_params=pltpu.CompilerParams(needs_layout_passes=False,
                                             use_tc_tiling_on_sc=False),
    )(_body)
    partial_sums = run(pos_h, pos_t, pos_r, neg_h, neg_t, neg_r,
                       ent_p, rel_p)
    return jnp.sum(partial_sums)


# speedup vs baseline: 2.5630x; 1.1481x over previous
import functools

import jax
import jax.numpy as jnp
from jax import lax
from jax.experimental import pallas as pl
from jax.experimental.pallas import tpu as pltpu
from jax.experimental.pallas import tpu_sc as plsc

_BATCH = 16384
_D = 64
_NC = 2
_NS = 16
_NW = _NC * _NS
_PER_W = _BATCH // _NW
_C = 64
_NCHUNK = _PER_W // _C
_NG = _C // 16
_MARGIN = 1.0
_PACK_W = 8192


def _pack_body(x, o):
    ident = jnp.eye(128, dtype=jnp.float32)
    ones = jnp.ones((1, _D), dtype=jnp.float32)
    red = (((1,), (0,)), ((), ()))
    dn = (((0,), (0,)), ((), ()))
    for u in range(x.shape[1] // 256):
        a = x[:, u * 256:u * 256 + 128]
        b = x[:, u * 256 + 128:u * 256 + 256]
        inva = lax.rsqrt(jnp.maximum(
            lax.dot_general(ones, a * a, red,
                            preferred_element_type=jnp.float32), 1e-12))
        invb = lax.rsqrt(jnp.maximum(
            lax.dot_general(ones, b * b, red,
                            preferred_element_type=jnp.float32), 1e-12))
        stacked = jnp.concatenate([a * inva, b * invb], axis=0)
        o[u * 128:(u + 1) * 128, :] = lax.dot_general(
            stacked, ident, dn, preferred_element_type=jnp.float32)


def _pack(table_t, w):
    nblk = (table_t.shape[1] + w - 1) // w
    return pl.pallas_call(
        _pack_body,
        grid=(nblk,),
        in_specs=[pl.BlockSpec((_D, w), lambda j: (0, j))],
        out_specs=pl.BlockSpec((w // 2, 128), lambda j: (j, 0)),
        out_shape=jax.ShapeDtypeStruct((nblk * w // 2, 128), jnp.float32),
    )(table_t)


def _row_of(i):
    return jnp.left_shift(jnp.right_shift(i, 8), 7) + jnp.bitwise_and(i, 127)


def _off_of(i):
    return jnp.left_shift(jnp.bitwise_and(jnp.right_shift(i, 7), 1), 6)


def _body(pos_h, pos_t, pos_r, neg_h, neg_t, neg_r, ent_p, rel_p, out,
          i_ph, i_pt, i_pr, i_nh, i_nt, i_nr,
          a_ph, a_pt, a_pr, a_nh, a_nt, a_nr,
          rA_ph, rA_pt, rA_pr, rA_nh, rA_nt, rA_nr,
          rB_ph, rB_pt, rB_pr, rB_nh, rB_nt, rB_nr,
          acc_v, semA, semB):
    wid = lax.axis_index("s") * _NC + lax.axis_index("c")
    iota = lax.iota(jnp.int32, 16)
    raws = (i_ph, i_pt, i_pr, i_nh, i_nt, i_nr)
    adjs = (a_ph, a_pt, a_pr, a_nh, a_nt, a_nr)
    bufs = ((rA_ph, rA_pt, rA_pr, rA_nh, rA_nt, rA_nr),
            (rB_ph, rB_pt, rB_pr, rB_nh, rB_nt, rB_nr))
    sems = (semA, semB)

    idx_cps = [
        pltpu.async_copy(src.at[pl.ds(wid * _PER_W, _PER_W)], raw_ref, semA)
        for src, raw_ref in zip((pos_h, pos_t, pos_r, neg_h, neg_t, neg_r),
                                raws)
    ]
    for cp in idx_cps:
        cp.wait()

    def remap(raw_ref, adj_ref):
        def step(v, _):
            x = raw_ref[pl.ds(v * 16, 16)]
            adj_ref[pl.ds(v * 16, 16)] = _row_of(x)
            return 0

        lax.fori_loop(0, _PER_W // 16, step, 0)

    for raw_ref, adj_ref in zip(raws, adjs):
        remap(raw_ref, adj_ref)

    def fire(k):
        sl = pl.ds(k * _C, _C)
        b = bufs[k % 2]
        sem = sems[k % 2]
        return [
            pltpu.async_copy(ent_p.at[a_ph.at[sl]], b[0], sem),
            pltpu.async_copy(ent_p.at[a_pt.at[sl]], b[1], sem),
            pltpu.async_copy(rel_p.at[a_pr.at[sl]], b[2], sem),
            pltpu.async_copy(ent_p.at[a_nh.at[sl]], b[3], sem),
            pltpu.async_copy(ent_p.at[a_nt.at[sl]], b[4], sem),
            pltpu.async_copy(rel_p.at[a_nr.at[sl]], b[5], sem),
        ]

    def make_score_step(b):
        def score_step(d, carry):
            pacc, nacc, row, offs = carry
            h = plsc.load_gather(b[0], [row, offs[0] + d])
            t = plsc.load_gather(b[1], [row, offs[1] + d])
            r = plsc.load_gather(b[2], [row, offs[2] + d])
            pacc = pacc + jnp.abs(h + r - t)
            h = plsc.load_gather(b[3], [row, offs[3] + d])
            t = plsc.load_gather(b[4], [row, offs[4] + d])
            r = plsc.load_gather(b[5], [row, offs[5] + d])
            nacc = nacc + jnp.abs(h + r - t)
            return pacc, nacc, row, offs

        return score_step

    def compute(k, total):
        b = bufs[k % 2]
        step = make_score_step(b)

        def group_body(g, tot):
            row = g * 16 + iota
            offs = tuple(_off_of(raw_ref[pl.ds(k * _C + g * 16, 16)])
                         for raw_ref in raws)
            z = jnp.zeros((16,), jnp.float32)
            pacc, nacc, _, _ = lax.fori_loop(
                0, _D, step, (z, z, row, offs), unroll=8)
            return tot + jnp.maximum(pacc - nacc + _MARGIN, 0.0)

        return lax.fori_loop(0, _NG, group_body, total)

    total = jnp.zeros((16,), jnp.float32)
    cps = fire(0)
    for k in range(_NCHUNK):
        nxt = fire(k + 1) if k + 1 < _NCHUNK else []
        for cp in cps:
            cp.wait()
        total = compute(k, total)
        cps = nxt

    acc_v[...] = total * (1.0 / _BATCH)
    pltpu.sync_copy(acc_v, out.at[wid])


@jax.jit
def kernel(pos_h, pos_t, pos_r, neg_h, neg_t, neg_r, ent_emb, rel_emb):
    ent_p = _pack(jnp.transpose(ent_emb), _PACK_W)
    rel_p = _pack(jnp.transpose(rel_emb), 1024)
    mesh = plsc.VectorSubcoreMesh(core_axis_name="c", subcore_axis_name="s",
                                  num_cores=_NC, num_subcores=_NS)
    run = functools.partial(
        pl.kernel,
        out_type=jax.ShapeDtypeStruct((_NW, 16), jnp.float32),
        mesh=mesh,
        scratch_types=[pltpu.VMEM((_PER_W,), jnp.int32)] * 12
        + [pltpu.VMEM((_C, 128), jnp.float32)] * 12
        + [pltpu.VMEM((16,), jnp.float32),
           pltpu.SemaphoreType.DMA, pltpu.SemaphoreType.DMA],
        compiler_params=pltpu.CompilerParams(needs_layout_passes=False,
                                             use_tc_tiling_on_sc=False),
    )(_body)
    partial_sums = run(pos_h, pos_t, pos_r, neg_h, neg_t, neg_r,
                       ent_p, rel_p)
    return jnp.sum(partial_sums)


# speedup vs baseline: 2.9200x; 1.1393x over previous
import functools

import jax
import jax.numpy as jnp
from jax import lax
from jax.experimental import pallas as pl
from jax.experimental.pallas import tpu as pltpu
from jax.experimental.pallas import tpu_sc as plsc

_BATCH = 16384
_D = 64
_NC = 2
_NS = 16
_NW = _NC * _NS
_PER_W = _BATCH // _NW
_C = 64
_NCHUNK = _PER_W // _C
_NG = _C // 16
_MARGIN = 1.0
_PACK_W = 16384


def _pack_body(x, o):
    ident = jnp.eye(128, dtype=jnp.float32)
    ones = jnp.ones((1, _D), dtype=jnp.float32)
    red = (((1,), (0,)), ((), ()))
    dn = (((0,), (0,)), ((), ()))
    for u in range(x.shape[1] // 256):
        a = x[:, u * 256:u * 256 + 128]
        b = x[:, u * 256 + 128:u * 256 + 256]
        inva = lax.rsqrt(jnp.maximum(
            lax.dot_general(ones, a * a, red,
                            preferred_element_type=jnp.float32), 1e-12))
        invb = lax.rsqrt(jnp.maximum(
            lax.dot_general(ones, b * b, red,
                            preferred_element_type=jnp.float32), 1e-12))
        stacked = jnp.concatenate([a * inva, b * invb], axis=0)
        o[u * 128:(u + 1) * 128, :] = lax.dot_general(
            stacked, ident, dn, preferred_element_type=jnp.float32)


def _pack(table_t, w):
    nblk = (table_t.shape[1] + w - 1) // w
    return pl.pallas_call(
        _pack_body,
        grid=(nblk,),
        in_specs=[pl.BlockSpec((_D, w), lambda j: (0, j))],
        out_specs=pl.BlockSpec((w // 2, 128), lambda j: (j, 0)),
        out_shape=jax.ShapeDtypeStruct((nblk * w // 2, 128), jnp.float32),
    )(table_t)


def _row_of(i):
    return jnp.left_shift(jnp.right_shift(i, 8), 7) + jnp.bitwise_and(i, 127)


def _off_of(i):
    return jnp.left_shift(jnp.bitwise_and(jnp.right_shift(i, 7), 1), 6)


def _body(pos_h, pos_t, pos_r, neg_h, neg_t, neg_r, ent_p, rel_p, out,
          i_ph, i_pt, i_pr, i_nh, i_nt, i_nr,
          a_ph, a_pt, a_pr, a_nh, a_nt, a_nr,
          rA_ph, rA_pt, rA_pr, rA_nh, rA_nt, rA_nr,
          rB_ph, rB_pt, rB_pr, rB_nh, rB_nt, rB_nr,
          acc_v, semA, semB):
    wid = lax.axis_index("s") * _NC + lax.axis_index("c")
    iota = lax.iota(jnp.int32, 16)
    raws = (i_ph, i_pt, i_pr, i_nh, i_nt, i_nr)
    adjs = (a_ph, a_pt, a_pr, a_nh, a_nt, a_nr)
    bufs = ((rA_ph, rA_pt, rA_pr, rA_nh, rA_nt, rA_nr),
            (rB_ph, rB_pt, rB_pr, rB_nh, rB_nt, rB_nr))
    sems = (semA, semB)

    idx_cps = [
        pltpu.async_copy(src.at[pl.ds(wid * _PER_W, _PER_W)], raw_ref, semA)
        for src, raw_ref in zip((pos_h, pos_t, pos_r, neg_h, neg_t, neg_r),
                                raws)
    ]
    for cp in idx_cps:
        cp.wait()

    def remap(raw_ref, adj_ref):
        def step(v, _):
            x = raw_ref[pl.ds(v * 16, 16)]
            adj_ref[pl.ds(v * 16, 16)] = _row_of(x)
            return 0

        lax.fori_loop(0, _PER_W // 16, step, 0)

    for raw_ref, adj_ref in zip(raws, adjs):
        remap(raw_ref, adj_ref)

    def fire(k):
        sl = pl.ds(k * _C, _C)
        b = bufs[k % 2]
        sem = sems[k % 2]
        return [
            pltpu.async_copy(ent_p.at[a_ph.at[sl]], b[0], sem),
            pltpu.async_copy(ent_p.at[a_pt.at[sl]], b[1], sem),
            pltpu.async_copy(rel_p.at[a_pr.at[sl]], b[2], sem),
            pltpu.async_copy(ent_p.at[a_nh.at[sl]], b[3], sem),
            pltpu.async_copy(ent_p.at[a_nt.at[sl]], b[4], sem),
            pltpu.async_copy(rel_p.at[a_nr.at[sl]], b[5], sem),
        ]

    def make_score_step(b):
        def score_step(d, carry):
            pacc, nacc, row, offs = carry
            h = plsc.load_gather(b[0], [row, offs[0] + d])
            t = plsc.load_gather(b[1], [row, offs[1] + d])
            r = plsc.load_gather(b[2], [row, offs[2] + d])
            pacc = pacc + jnp.abs(h + r - t)
            h = plsc.load_gather(b[3], [row, offs[3] + d])
            t = plsc.load_gather(b[4], [row, offs[4] + d])
            r = plsc.load_gather(b[5], [row, offs[5] + d])
            nacc = nacc + jnp.abs(h + r - t)
            return pacc, nacc, row, offs

        return score_step

    def compute(k, total):
        b = bufs[k % 2]
        step = make_score_step(b)

        def group_body(g, tot):
            row = g * 16 + iota
            offs = tuple(_off_of(raw_ref[pl.ds(k * _C + g * 16, 16)])
                         for raw_ref in raws)
            z = jnp.zeros((16,), jnp.float32)
            pacc, nacc, _, _ = lax.fori_loop(
                0, _D, step, (z, z, row, offs), unroll=8)
            return tot + jnp.maximum(pacc - nacc + _MARGIN, 0.0)

        return lax.fori_loop(0, _NG, group_body, total)

    total = jnp.zeros((16,), jnp.float32)
    cps = fire(0)
    for k in range(_NCHUNK):
        nxt = fire(k + 1) if k + 1 < _NCHUNK else []
        for cp in cps:
            cp.wait()
        total = compute(k, total)
        cps = nxt

    acc_v[...] = total * (1.0 / _BATCH)
    pltpu.sync_copy(acc_v, out.at[wid])


@jax.jit
def kernel(pos_h, pos_t, pos_r, neg_h, neg_t, neg_r, ent_emb, rel_emb):
    ent_p = _pack(jnp.transpose(ent_emb), _PACK_W)
    rel_p = _pack(jnp.transpose(rel_emb), 1024)
    mesh = plsc.VectorSubcoreMesh(core_axis_name="c", subcore_axis_name="s",
                                  num_cores=_NC, num_subcores=_NS)
    run = functools.partial(
        pl.kernel,
        out_type=jax.ShapeDtypeStruct((_NW, 16), jnp.float32),
        mesh=mesh,
        scratch_types=[pltpu.VMEM((_PER_W,), jnp.int32)] * 12
        + [pltpu.VMEM((_C, 128), jnp.float32)] * 12
        + [pltpu.VMEM((16,), jnp.float32),
           pltpu.SemaphoreType.DMA, pltpu.SemaphoreType.DMA],
        compiler_params=pltpu.CompilerParams(needs_layout_passes=False,
                                             use_tc_tiling_on_sc=False),
    )(_body)
    partial_sums = run(pos_h, pos_t, pos_r, neg_h, neg_t, neg_r,
                       ent_p, rel_p)
    return jnp.sum(partial_sums)


# speedup vs baseline: 3.0032x; 1.0285x over previous
import functools

import jax
import jax.numpy as jnp
from jax import lax
from jax.experimental import pallas as pl
from jax.experimental.pallas import tpu as pltpu
from jax.experimental.pallas import tpu_sc as plsc

_BATCH = 16384
_D = 64
_NC = 2
_NS = 16
_NW = _NC * _NS
_PER_W = _BATCH // _NW
_C = 64
_NCHUNK = _PER_W // _C
_NG = _C // 16
_MARGIN = 1.0
_PACK_W = 32768


def _pack_body(x, o):
    ident = jnp.eye(128, dtype=jnp.float32)
    ones = jnp.ones((1, _D), dtype=jnp.float32)
    red = (((1,), (0,)), ((), ()))
    dn = (((0,), (0,)), ((), ()))
    for u in range(x.shape[1] // 256):
        a = x[:, u * 256:u * 256 + 128]
        b = x[:, u * 256 + 128:u * 256 + 256]
        inva = lax.rsqrt(jnp.maximum(
            lax.dot_general(ones, a * a, red,
                            preferred_element_type=jnp.float32), 1e-12))
        invb = lax.rsqrt(jnp.maximum(
            lax.dot_general(ones, b * b, red,
                            preferred_element_type=jnp.float32), 1e-12))
        stacked = jnp.concatenate([a * inva, b * invb], axis=0)
        o[u * 128:(u + 1) * 128, :] = lax.dot_general(
            stacked, ident, dn, preferred_element_type=jnp.float32)


def _pack(table_t, w):
    nblk = (table_t.shape[1] + w - 1) // w
    return pl.pallas_call(
        _pack_body,
        grid=(nblk,),
        in_specs=[pl.BlockSpec((_D, w), lambda j: (0, j))],
        out_specs=pl.BlockSpec((w // 2, 128), lambda j: (j, 0)),
        out_shape=jax.ShapeDtypeStruct((nblk * w // 2, 128), jnp.float32),
    )(table_t)


def _row_of(i):
    return jnp.left_shift(jnp.right_shift(i, 8), 7) + jnp.bitwise_and(i, 127)


def _off_of(i):
    return jnp.left_shift(jnp.bitwise_and(jnp.right_shift(i, 7), 1), 6)


def _body(pos_h, pos_t, pos_r, neg_h, neg_t, neg_r, ent_p, rel_p, out,
          i_ph, i_pt, i_pr, i_nh, i_nt, i_nr,
          a_ph, a_pt, a_pr, a_nh, a_nt, a_nr,
          rA_ph, rA_pt, rA_pr, rA_nh, rA_nt, rA_nr,
          rB_ph, rB_pt, rB_pr, rB_nh, rB_nt, rB_nr,
          acc_v, semA, semB):
    wid = lax.axis_index("s") * _NC + lax.axis_index("c")
    iota = lax.iota(jnp.int32, 16)
    raws = (i_ph, i_pt, i_pr, i_nh, i_nt, i_nr)
    adjs = (a_ph, a_pt, a_pr, a_nh, a_nt, a_nr)
    bufs = ((rA_ph, rA_pt, rA_pr, rA_nh, rA_nt, rA_nr),
            (rB_ph, rB_pt, rB_pr, rB_nh, rB_nt, rB_nr))
    sems = (semA, semB)

    idx_cps = [
        pltpu.async_copy(src.at[pl.ds(wid * _PER_W, _PER_W)], raw_ref, semA)
        for src, raw_ref in zip((pos_h, pos_t, pos_r, neg_h, neg_t, neg_r),
                                raws)
    ]
    for cp in idx_cps:
        cp.wait()

    def remap(raw_ref, adj_ref):
        def step(v, _):
            x = raw_ref[pl.ds(v * 16, 16)]
            adj_ref[pl.ds(v * 16, 16)] = _row_of(x)
            return 0

        lax.fori_loop(0, _PER_W // 16, step, 0)

    for raw_ref, adj_ref in zip(raws, adjs):
        remap(raw_ref, adj_ref)

    def fire(k):
        sl = pl.ds(k * _C, _C)
        b = bufs[k % 2]
        sem = sems[k % 2]
        return [
            pltpu.async_copy(ent_p.at[a_ph.at[sl]], b[0], sem),
            pltpu.async_copy(ent_p.at[a_pt.at[sl]], b[1], sem),
            pltpu.async_copy(rel_p.at[a_pr.at[sl]], b[2], sem),
            pltpu.async_copy(ent_p.at[a_nh.at[sl]], b[3], sem),
            pltpu.async_copy(ent_p.at[a_nt.at[sl]], b[4], sem),
            pltpu.async_copy(rel_p.at[a_nr.at[sl]], b[5], sem),
        ]

    def make_score_step(b):
        def score_step(d, carry):
            pacc, nacc, row, offs = carry
            h = plsc.load_gather(b[0], [row, offs[0] + d])
            t = plsc.load_gather(b[1], [row, offs[1] + d])
            r = plsc.load_gather(b[2], [row, offs[2] + d])
            pacc = pacc + jnp.abs(h + r - t)
            h = plsc.load_gather(b[3], [row, offs[3] + d])
            t = plsc.load_gather(b[4], [row, offs[4] + d])
            r = plsc.load_gather(b[5], [row, offs[5] + d])
            nacc = nacc + jnp.abs(h + r - t)
            return pacc, nacc, row, offs

        return score_step

    def compute(k, total):
        b = bufs[k % 2]
        step = make_score_step(b)

        def group_body(g, tot):
            row = g * 16 + iota
            offs = tuple(_off_of(raw_ref[pl.ds(k * _C + g * 16, 16)])
                         for raw_ref in raws)
            z = jnp.zeros((16,), jnp.float32)
            pacc, nacc, _, _ = lax.fori_loop(
                0, _D, step, (z, z, row, offs), unroll=8)
            return tot + jnp.maximum(pacc - nacc + _MARGIN, 0.0)

        return lax.fori_loop(0, _NG, group_body, total)

    total = jnp.zeros((16,), jnp.float32)
    cps = fire(0)
    for k in range(_NCHUNK):
        nxt = fire(k + 1) if k + 1 < _NCHUNK else []
        for cp in cps:
            cp.wait()
        total = compute(k, total)
        cps = nxt

    acc_v[...] = total * (1.0 / _BATCH)
    pltpu.sync_copy(acc_v, out.at[wid])


@jax.jit
def kernel(pos_h, pos_t, pos_r, neg_h, neg_t, neg_r, ent_emb, rel_emb):
    ent_p = _pack(jnp.transpose(ent_emb), _PACK_W)
    rel_p = _pack(jnp.transpose(rel_emb), 1024)
    mesh = plsc.VectorSubcoreMesh(core_axis_name="c", subcore_axis_name="s",
                                  num_cores=_NC, num_subcores=_NS)
    run = functools.partial(
        pl.kernel,
        out_type=jax.ShapeDtypeStruct((_NW, 16), jnp.float32),
        mesh=mesh,
        scratch_types=[pltpu.VMEM((_PER_W,), jnp.int32)] * 12
        + [pltpu.VMEM((_C, 128), jnp.float32)] * 12
        + [pltpu.VMEM((16,), jnp.float32),
           pltpu.SemaphoreType.DMA, pltpu.SemaphoreType.DMA],
        compiler_params=pltpu.CompilerParams(needs_layout_passes=False,
                                             use_tc_tiling_on_sc=False),
    )(_body)
    partial_sums = run(pos_h, pos_t, pos_r, neg_h, neg_t, neg_r,
                       ent_p, rel_p)
    return jnp.sum(partial_sums)
